# baseline (speedup 1.0000x reference)
import jax
import jax.numpy as jnp
from jax.experimental import pallas as pl
from jax.experimental.pallas import tpu as pltpu

N_E = 1024
E_DIM = 768
BETA = 0.25
TOK = 16 * 24 * 24
TILE = 512
NSTEP = TOK // TILE


def _vq_body(z_ref, cb_ref,
             d_ref, prob_ref, onehot_ref, idx_ref, zq_ref,
             loss_ref, losskl_ref, perp_ref,
             cbn_scr, embn_scr, ecount_scr, cossum_scr):
    step = pl.program_id(0)

    @pl.when(step == 0)
    def _init():
        cb = cb_ref[...]
        clip = cb[:, :512]
        dino = cb[:, 512:]
        nclip = jnp.sqrt(jnp.sum(clip * clip, axis=1, keepdims=True))
        ndino = jnp.sqrt(jnp.sum(dino * dino, axis=1, keepdims=True))
        cbn = jnp.concatenate([clip / nclip, dino / ndino], axis=1)
        cbn_scr[...] = cbn
        embn_scr[...] = jnp.sqrt(jnp.sum(cbn * cbn, axis=1))[None, :]
        ecount_scr[...] = jnp.zeros_like(ecount_scr)
        cossum_scr[...] = jnp.zeros_like(cossum_scr)

    cbn = cbn_scr[...]
    z = z_ref[...]
    znorm = jnp.sqrt(jnp.sum(z * z, axis=1, keepdims=True))
    logits = jax.lax.dot_general(
        z, cbn, (((1,), (1,)), ((), ())),
        preferred_element_type=jnp.float32)
    embn = embn_scr[...]
    d = logits / (znorm * embn + 1e-6)
    d_ref[...] = d

    rowmax = jnp.max(d, axis=1, keepdims=True)
    colids = jax.lax.broadcasted_iota(jnp.int32, d.shape, 1)
    idx = jnp.min(jnp.where(d == rowmax, colids, N_E), axis=1,
                  keepdims=True)
    idx_ref[...] = idx
    onehot = (colids == idx).astype(jnp.float32)
    onehot_ref[...] = onehot

    e = jnp.exp(d - rowmax)
    prob_ref[...] = e * (1.0 / jnp.sum(e, axis=1, keepdims=True))

    zq = jnp.dot(onehot, cbn, preferred_element_type=jnp.float32)
    zq_ref[...] = zq

    cos = jnp.sum(zq * z, axis=1, keepdims=True) * (
        1.0 / (1.4142135623730951 * jnp.maximum(znorm, 1e-8)))
    cossum_scr[...] = cossum_scr[...] + jnp.sum(cos, keepdims=True)
    ecount_scr[...] += jnp.sum(onehot, axis=0)[None, :]

    @pl.when(step == NSTEP - 1)
    def _fin():
        mean_cos = cossum_scr[...] / TOK
        loss_ref[...] = (1.0 - mean_cos) + BETA * (1.0 - mean_cos)
        e_mean = ecount_scr[...] / TOK
        losskl_ref[...] = -jnp.sum(
            e_mean * jnp.log((1.0 / N_E) / (e_mean + 1e-6)),
            axis=1, keepdims=True)
        perp_ref[...] = jnp.exp(
            -jnp.sum(e_mean * jnp.log(e_mean + 1e-6), axis=1, keepdims=True))


def kernel(z, embedding_weight):
    zf = z.reshape(TOK, E_DIM)
    out_shapes = (
        jax.ShapeDtypeStruct((TOK, N_E), jnp.float32),
        jax.ShapeDtypeStruct((TOK, N_E), jnp.float32),
        jax.ShapeDtypeStruct((TOK, N_E), jnp.float32),
        jax.ShapeDtypeStruct((TOK, 1), jnp.int32),
        jax.ShapeDtypeStruct((TOK, E_DIM), jnp.float32),
        jax.ShapeDtypeStruct((1, 1), jnp.float32),
        jax.ShapeDtypeStruct((1, 1), jnp.float32),
        jax.ShapeDtypeStruct((1, 1), jnp.float32),
    )
    big = pl.BlockSpec((TILE, N_E), lambda i: (i, 0))
    scalar = pl.BlockSpec((1, 1), lambda i: (0, 0))
    d, prob, onehot, idx, zq_st, loss, loss_kl, perp = pl.pallas_call(
        _vq_body,
        grid=(NSTEP,),
        in_specs=[
            pl.BlockSpec((TILE, E_DIM), lambda i: (i, 0)),
            pl.BlockSpec((N_E, E_DIM), lambda i: (0, 0)),
        ],
        out_specs=(
            big, big, big,
            pl.BlockSpec((TILE, 1), lambda i: (i, 0)),
            pl.BlockSpec((TILE, E_DIM), lambda i: (i, 0)),
            scalar, scalar, scalar,
        ),
        out_shape=out_shapes,
        scratch_shapes=[
            pltpu.VMEM((N_E, E_DIM), jnp.float32),
            pltpu.VMEM((1, N_E), jnp.float32),
            pltpu.VMEM((1, N_E), jnp.float32),
            pltpu.VMEM((1, 1), jnp.float32),
        ],
    )(zf, embedding_weight)

    loss = loss[0, 0]
    loss_kl = loss_kl[0, 0]
    perplexity = perp[0, 0]
    constrative_loss = jnp.asarray(0.0, dtype=jnp.float32)
    z_q_st = zq_st.reshape(z.shape)
    return (loss, constrative_loss, loss_kl, prob, d, z_q_st,
            perplexity, onehot, idx)


# speedup vs baseline: 1.0492x; 1.0492x over previous
import jax
import jax.numpy as jnp
from jax.experimental import pallas as pl
from jax.experimental.pallas import tpu as pltpu

N_E = 1024
E_DIM = 768
BETA = 0.25
TOK = 16 * 24 * 24
TILE = 512
NSTEP = TOK // TILE


def _vq_body(z_ref, cb_ref,
             d_ref, prob_ref, onehot_ref, idx_ref, zq_ref,
             loss_ref, losskl_ref, perp_ref,
             cbn_scr, embn_scr, ecount_scr, cossum_scr):
    step = pl.program_id(0)

    @pl.when(step == 0)
    def _init():
        cb = cb_ref[...]
        clip = cb[:, :512]
        dino = cb[:, 512:]
        nclip = jnp.sqrt(jnp.sum(clip * clip, axis=1, keepdims=True))
        ndino = jnp.sqrt(jnp.sum(dino * dino, axis=1, keepdims=True))
        cbn = jnp.concatenate([clip / nclip, dino / ndino], axis=1)
        cbn_scr[...] = cbn
        embn_scr[...] = jnp.sqrt(jnp.sum(cbn * cbn, axis=1))[None, :]
        ecount_scr[...] = jnp.zeros_like(ecount_scr)
        cossum_scr[...] = jnp.zeros_like(cossum_scr)

    cbn = cbn_scr[...]
    z = z_ref[...]
    znorm = jnp.sqrt(jnp.sum(z * z, axis=1, keepdims=True))
    logits = jax.lax.dot_general(
        z, cbn, (((1,), (1,)), ((), ())),
        preferred_element_type=jnp.float32)
    embn = embn_scr[...]
    d = logits / (znorm * embn + 1e-6)
    d_ref[...] = d

    rowmax = jnp.max(d, axis=1, keepdims=True)
    colids = jax.lax.broadcasted_iota(jnp.int32, d.shape, 1)
    idx = jnp.min(jnp.where(d == rowmax, colids, N_E), axis=1,
                  keepdims=True)
    idx_ref[...] = idx.reshape(1, 1, TILE)
    onehot = (colids == idx).astype(jnp.float32)
    onehot_ref[...] = onehot

    e = jnp.exp(d - rowmax)
    prob_ref[...] = e * (1.0 / jnp.sum(e, axis=1, keepdims=True))

    zq = jnp.dot(onehot, cbn, preferred_element_type=jnp.float32)
    zq_ref[...] = zq

    cos = jnp.sum(zq * z, axis=1, keepdims=True) * (
        1.0 / (1.4142135623730951 * jnp.maximum(znorm, 1e-8)))
    cossum_scr[...] = cossum_scr[...] + jnp.sum(cos, keepdims=True)
    ecount_scr[...] += jnp.sum(onehot, axis=0)[None, :]

    @pl.when(step == NSTEP - 1)
    def _fin():
        mean_cos = cossum_scr[...] / TOK
        loss_ref[...] = (1.0 - mean_cos) + BETA * (1.0 - mean_cos)
        e_mean = ecount_scr[...] / TOK
        losskl_ref[...] = -jnp.sum(
            e_mean * jnp.log((1.0 / N_E) / (e_mean + 1e-6)),
            axis=1, keepdims=True)
        perp_ref[...] = jnp.exp(
            -jnp.sum(e_mean * jnp.log(e_mean + 1e-6), axis=1, keepdims=True))


def kernel(z, embedding_weight):
    zf = z.reshape(TOK, E_DIM)
    out_shapes = (
        jax.ShapeDtypeStruct((TOK, N_E), jnp.float32),
        jax.ShapeDtypeStruct((TOK, N_E), jnp.float32),
        jax.ShapeDtypeStruct((TOK, N_E), jnp.float32),
        jax.ShapeDtypeStruct((NSTEP, 1, TILE), jnp.int32),
        jax.ShapeDtypeStruct((TOK, E_DIM), jnp.float32),
        jax.ShapeDtypeStruct((1, 1), jnp.float32),
        jax.ShapeDtypeStruct((1, 1), jnp.float32),
        jax.ShapeDtypeStruct((1, 1), jnp.float32),
    )
    big = pl.BlockSpec((TILE, N_E), lambda i: (i, 0))
    scalar = pl.BlockSpec((1, 1), lambda i: (0, 0))
    d, prob, onehot, idx, zq_st, loss, loss_kl, perp = pl.pallas_call(
        _vq_body,
        grid=(NSTEP,),
        in_specs=[
            pl.BlockSpec((TILE, E_DIM), lambda i: (i, 0)),
            pl.BlockSpec((N_E, E_DIM), lambda i: (0, 0)),
        ],
        out_specs=(
            big, big, big,
            pl.BlockSpec((1, 1, TILE), lambda i: (i, 0, 0)),
            pl.BlockSpec((TILE, E_DIM), lambda i: (i, 0)),
            scalar, scalar, scalar,
        ),
        out_shape=out_shapes,
        scratch_shapes=[
            pltpu.VMEM((N_E, E_DIM), jnp.float32),
            pltpu.VMEM((1, N_E), jnp.float32),
            pltpu.VMEM((1, N_E), jnp.float32),
            pltpu.VMEM((1, 1), jnp.float32),
        ],
    )(zf, embedding_weight)

    loss = loss[0, 0]
    loss_kl = loss_kl[0, 0]
    perplexity = perp[0, 0]
    constrative_loss = jnp.asarray(0.0, dtype=jnp.float32)
    z_q_st = zq_st.reshape(z.shape)
    idx = idx.reshape(TOK, 1)
    return (loss, constrative_loss, loss_kl, prob, d, z_q_st,
            perplexity, onehot, idx)


# speedup vs baseline: 1.0623x; 1.0125x over previous
import jax
import jax.numpy as jnp
from jax.experimental import pallas as pl
from jax.experimental.pallas import tpu as pltpu

N_E = 1024
E_DIM = 768
BETA = 0.25
TOK = 16 * 24 * 24
TILE = 1024
NSTEP = TOK // TILE


def _vq_body(z_ref, cb_ref,
             d_ref, prob_ref, onehot_ref, idx_ref, zq_ref,
             loss_ref, losskl_ref, perp_ref,
             cbn_scr, embn_scr, ecount_scr, cossum_scr):
    step = pl.program_id(0)

    @pl.when(step == 0)
    def _init():
        cb = cb_ref[...]
        clip = cb[:, :512]
        dino = cb[:, 512:]
        nclip = jnp.sqrt(jnp.sum(clip * clip, axis=1, keepdims=True))
        ndino = jnp.sqrt(jnp.sum(dino * dino, axis=1, keepdims=True))
        cbn = jnp.concatenate([clip / nclip, dino / ndino], axis=1)
        cbn_scr[...] = cbn
        embn_scr[...] = jnp.sqrt(jnp.sum(cbn * cbn, axis=1))[None, :]
        ecount_scr[...] = jnp.zeros_like(ecount_scr)
        cossum_scr[...] = jnp.zeros_like(cossum_scr)

    cbn = cbn_scr[...]
    z = z_ref[...]
    znorm = jnp.sqrt(jnp.sum(z * z, axis=1, keepdims=True))
    logits = jax.lax.dot_general(
        z, cbn, (((1,), (1,)), ((), ())),
        preferred_element_type=jnp.float32)
    embn = embn_scr[...]
    d = logits / (znorm * embn + 1e-6)
    d_ref[...] = d

    rowmax = jnp.max(d, axis=1, keepdims=True)
    colids = jax.lax.broadcasted_iota(jnp.int32, d.shape, 1)
    idx = jnp.min(jnp.where(d == rowmax, colids, N_E), axis=1,
                  keepdims=True)
    idx_ref[...] = idx.reshape(1, 1, TILE)
    onehot = (colids == idx).astype(jnp.float32)
    onehot_ref[...] = onehot

    e = jnp.exp(d - rowmax)
    prob_ref[...] = e * (1.0 / jnp.sum(e, axis=1, keepdims=True))

    zq = jnp.dot(onehot, cbn, preferred_element_type=jnp.float32)
    zq_ref[...] = zq

    cos = jnp.sum(zq * z, axis=1, keepdims=True) * (
        1.0 / (1.4142135623730951 * jnp.maximum(znorm, 1e-8)))
    cossum_scr[...] = cossum_scr[...] + jnp.sum(cos, keepdims=True)
    ecount_scr[...] += jnp.sum(onehot, axis=0)[None, :]

    @pl.when(step == NSTEP - 1)
    def _fin():
        mean_cos = cossum_scr[...] / TOK
        loss_ref[...] = (1.0 - mean_cos) + BETA * (1.0 - mean_cos)
        e_mean = ecount_scr[...] / TOK
        losskl_ref[...] = -jnp.sum(
            e_mean * jnp.log((1.0 / N_E) / (e_mean + 1e-6)),
            axis=1, keepdims=True)
        perp_ref[...] = jnp.exp(
            -jnp.sum(e_mean * jnp.log(e_mean + 1e-6), axis=1, keepdims=True))


def kernel(z, embedding_weight):
    zf = z.reshape(TOK, E_DIM)
    out_shapes = (
        jax.ShapeDtypeStruct((TOK, N_E), jnp.float32),
        jax.ShapeDtypeStruct((TOK, N_E), jnp.float32),
        jax.ShapeDtypeStruct((TOK, N_E), jnp.float32),
        jax.ShapeDtypeStruct((NSTEP, 1, TILE), jnp.int32),
        jax.ShapeDtypeStruct((TOK, E_DIM), jnp.float32),
        jax.ShapeDtypeStruct((1, 1), jnp.float32),
        jax.ShapeDtypeStruct((1, 1), jnp.float32),
        jax.ShapeDtypeStruct((1, 1), jnp.float32),
    )
    big = pl.BlockSpec((TILE, N_E), lambda i: (i, 0))
    scalar = pl.BlockSpec((1, 1), lambda i: (0, 0))
    d, prob, onehot, idx, zq_st, loss, loss_kl, perp = pl.pallas_call(
        _vq_body,
        grid=(NSTEP,),
        in_specs=[
            pl.BlockSpec((TILE, E_DIM), lambda i: (i, 0)),
            pl.BlockSpec((N_E, E_DIM), lambda i: (0, 0)),
        ],
        out_specs=(
            big, big, big,
            pl.BlockSpec((1, 1, TILE), lambda i: (i, 0, 0)),
            pl.BlockSpec((TILE, E_DIM), lambda i: (i, 0)),
            scalar, scalar, scalar,
        ),
        out_shape=out_shapes,
        scratch_shapes=[
            pltpu.VMEM((N_E, E_DIM), jnp.float32),
            pltpu.VMEM((1, N_E), jnp.float32),
            pltpu.VMEM((1, N_E), jnp.float32),
            pltpu.VMEM((1, 1), jnp.float32),
        ],
    )(zf, embedding_weight)

    loss = loss[0, 0]
    loss_kl = loss_kl[0, 0]
    perplexity = perp[0, 0]
    constrative_loss = jnp.asarray(0.0, dtype=jnp.float32)
    z_q_st = zq_st.reshape(z.shape)
    idx = idx.reshape(TOK, 1)
    return (loss, constrative_loss, loss_kl, prob, d, z_q_st,
            perplexity, onehot, idx)


# speedup vs baseline: 1.0930x; 1.0289x over previous
import jax
import jax.numpy as jnp
from jax.experimental import pallas as pl
from jax.experimental.pallas import tpu as pltpu

N_E = 1024
E_DIM = 768
BETA = 0.25
TOK = 16 * 24 * 24
TILE = 1024
NSTEP = TOK // TILE


def _vq_body(z_ref, cb_ref,
             d_ref, prob_ref, onehot_ref, idx_ref, zq_ref,
             loss_ref, losskl_ref, perp_ref,
             cbn_scr, embn_scr, embnc_scr, ecount_scr, cossum_scr):
    step = pl.program_id(0)

    @pl.when(step == 0)
    def _init():
        cb = cb_ref[...]
        clip = cb[:, :512]
        dino = cb[:, 512:]
        nclip = jnp.sqrt(jnp.sum(clip * clip, axis=1, keepdims=True))
        ndino = jnp.sqrt(jnp.sum(dino * dino, axis=1, keepdims=True))
        cbn = jnp.concatenate([clip / nclip, dino / ndino], axis=1)
        cbn_scr[...] = cbn
        embnc = jnp.sqrt(jnp.sum(cbn * cbn, axis=1, keepdims=True))
        embnc_scr[...] = embnc
        embn_scr[...] = embnc.reshape(1, N_E)
        ecount_scr[...] = jnp.zeros_like(ecount_scr)
        cossum_scr[...] = jnp.zeros_like(cossum_scr)

    cbn = cbn_scr[...]
    z = z_ref[...]
    znorm = jnp.sqrt(jnp.sum(z * z, axis=1, keepdims=True))
    logits = jax.lax.dot_general(
        z, cbn, (((1,), (1,)), ((), ())),
        preferred_element_type=jnp.float32)
    embn = embn_scr[...]
    d = logits / (znorm * embn + 1e-6)
    d_ref[...] = d

    rowmax = jnp.max(d, axis=1, keepdims=True)
    colids = jax.lax.broadcasted_iota(jnp.int32, d.shape, 1)
    idx = jnp.min(jnp.where(d == rowmax, colids, N_E), axis=1,
                  keepdims=True)
    idx_ref[...] = idx.reshape(1, 1, TILE)
    onehot = (colids == idx).astype(jnp.float32)
    onehot_ref[...] = onehot

    e = jnp.exp(d)
    prob_ref[...] = e * (1.0 / jnp.sum(e, axis=1, keepdims=True))

    zq = jnp.dot(onehot, cbn, preferred_element_type=jnp.float32)
    zq_ref[...] = zq

    embn_sel = jnp.dot(onehot, embnc_scr[...],
                       preferred_element_type=jnp.float32)
    cos = rowmax * (znorm * embn_sel + 1e-6) * (
        1.0 / (1.4142135623730951 * jnp.maximum(znorm, 1e-8)))
    cossum_scr[...] = cossum_scr[...] + jnp.sum(cos, keepdims=True)
    ones_row = jnp.ones((1, TILE), dtype=jnp.float32)
    ecount_scr[...] += jnp.dot(ones_row, onehot,
                               preferred_element_type=jnp.float32)

    @pl.when(step == NSTEP - 1)
    def _fin():
        mean_cos = cossum_scr[...] / TOK
        loss_ref[...] = (1.0 - mean_cos) + BETA * (1.0 - mean_cos)
        e_mean = ecount_scr[...] / TOK
        losskl_ref[...] = -jnp.sum(
            e_mean * jnp.log((1.0 / N_E) / (e_mean + 1e-6)),
            axis=1, keepdims=True)
        perp_ref[...] = jnp.exp(
            -jnp.sum(e_mean * jnp.log(e_mean + 1e-6), axis=1, keepdims=True))


def kernel(z, embedding_weight):
    zf = z.reshape(TOK, E_DIM)
    out_shapes = (
        jax.ShapeDtypeStruct((TOK, N_E), jnp.float32),
        jax.ShapeDtypeStruct((TOK, N_E), jnp.float32),
        jax.ShapeDtypeStruct((TOK, N_E), jnp.float32),
        jax.ShapeDtypeStruct((NSTEP, 1, TILE), jnp.int32),
        jax.ShapeDtypeStruct((TOK, E_DIM), jnp.float32),
        jax.ShapeDtypeStruct((1, 1), jnp.float32),
        jax.ShapeDtypeStruct((1, 1), jnp.float32),
        jax.ShapeDtypeStruct((1, 1), jnp.float32),
    )
    big = pl.BlockSpec((TILE, N_E), lambda i: (i, 0))
    scalar = pl.BlockSpec((1, 1), lambda i: (0, 0))
    d, prob, onehot, idx, zq_st, loss, loss_kl, perp = pl.pallas_call(
        _vq_body,
        grid=(NSTEP,),
        in_specs=[
            pl.BlockSpec((TILE, E_DIM), lambda i: (i, 0)),
            pl.BlockSpec((N_E, E_DIM), lambda i: (0, 0)),
        ],
        out_specs=(
            big, big, big,
            pl.BlockSpec((1, 1, TILE), lambda i: (i, 0, 0)),
            pl.BlockSpec((TILE, E_DIM), lambda i: (i, 0)),
            scalar, scalar, scalar,
        ),
        out_shape=out_shapes,
        scratch_shapes=[
            pltpu.VMEM((N_E, E_DIM), jnp.float32),
            pltpu.VMEM((1, N_E), jnp.float32),
            pltpu.VMEM((N_E, 1), jnp.float32),
            pltpu.VMEM((1, N_E), jnp.float32),
            pltpu.VMEM((1, 1), jnp.float32),
        ],
    )(zf, embedding_weight)

    loss = loss[0, 0]
    loss_kl = loss_kl[0, 0]
    perplexity = perp[0, 0]
    constrative_loss = jnp.asarray(0.0, dtype=jnp.float32)
    z_q_st = zq_st.reshape(z.shape)
    idx = idx.reshape(TOK, 1)
    return (loss, constrative_loss, loss_kl, prob, d, z_q_st,
            perplexity, onehot, idx)


# speedup vs baseline: 1.1473x; 1.0497x over previous
import jax
import jax.numpy as jnp
from jax.experimental import pallas as pl
from jax.experimental.pallas import tpu as pltpu

N_E = 1024
E_DIM = 768
BETA = 0.25
TOK = 16 * 24 * 24
TILE = 1024
NSTEP = TOK // TILE
AUG = E_DIM + 3
SQRT2 = 1.4142135623730951


def _vq_body(z_ref, cb_ref,
             d_ref, prob_ref, onehot_ref, idx_ref, zq_ref,
             loss_ref, losskl_ref, perp_ref,
             aug_scr, embn_scr, ecount_scr, cossum_scr):
    step = pl.program_id(0)

    @pl.when(step == 0)
    def _init():
        cb = cb_ref[...]
        clip = cb[:, :512]
        dino = cb[:, 512:]
        nclip = jnp.sqrt(jnp.sum(clip * clip, axis=1, keepdims=True))
        ndino = jnp.sqrt(jnp.sum(dino * dino, axis=1, keepdims=True))
        cbn = jnp.concatenate([clip / nclip, dino / ndino], axis=1)
        embnc = jnp.sqrt(jnp.sum(cbn * cbn, axis=1, keepdims=True))
        rowids = jax.lax.broadcasted_iota(
            jnp.int32, (N_E, 1), 0).astype(jnp.float32)
        ones_col = jnp.ones((N_E, 1), jnp.float32)
        aug_scr[...] = jnp.concatenate(
            [cbn, rowids, ones_col, embnc], axis=1)
        embn_scr[...] = embnc.reshape(1, N_E)
        ecount_scr[...] = jnp.zeros_like(ecount_scr)
        cossum_scr[...] = jnp.zeros_like(cossum_scr)

    cbn = aug_scr[:, :E_DIM]
    z = z_ref[...]
    znorm = jnp.sqrt(jnp.sum(z * z, axis=1, keepdims=True))
    logits = jax.lax.dot_general(
        z, cbn, (((1,), (1,)), ((), ())),
        preferred_element_type=jnp.float32)
    embn = embn_scr[...]
    d = logits / (znorm * embn + 1e-6)
    d_ref[...] = d

    rowmax = jnp.max(d, axis=1, keepdims=True)
    mask = d == rowmax
    onehot = mask.astype(jnp.float32)
    onehot_ref[...] = onehot

    e = jnp.exp(d)
    prob_ref[...] = e * (1.0 / jnp.sum(e, axis=1, keepdims=True))

    aug = jnp.dot(onehot, aug_scr[...],
                  preferred_element_type=jnp.float32)
    zq_ref[...] = aug[:, :E_DIM]
    idxf = aug[:, E_DIM:E_DIM + 1]
    cnt = aug[:, E_DIM + 1:E_DIM + 2]
    embn_sel = aug[:, E_DIM + 2:E_DIM + 3]
    idx_ref[...] = idxf.astype(jnp.int32).reshape(1, 1, TILE)

    cos = rowmax * (znorm * embn_sel + 1e-6) * (
        1.0 / (SQRT2 * jnp.maximum(znorm, 1e-8)))
    cos_delta = jnp.sum(cos, keepdims=True)
    cossum_scr[...] = cossum_scr[...] + cos_delta
    ones_row = jnp.ones((1, TILE), dtype=jnp.float32)
    hist_delta = jnp.dot(ones_row, onehot,
                         preferred_element_type=jnp.float32)
    ecount_scr[...] += hist_delta

    @pl.when(jnp.max(cnt) > 1.5)
    def _tie_fixup():
        colids = jax.lax.broadcasted_iota(jnp.int32, d.shape, 1)
        idx_e = jnp.min(jnp.where(mask, colids, N_E), axis=1,
                        keepdims=True)
        oh_e = (colids == idx_e).astype(jnp.float32)
        onehot_ref[...] = oh_e
        idx_ref[...] = idx_e.reshape(1, 1, TILE)
        aug_e = jnp.dot(oh_e, aug_scr[...],
                        preferred_element_type=jnp.float32)
        zq_ref[...] = aug_e[:, :E_DIM]
        embn_e = aug_e[:, E_DIM + 2:E_DIM + 3]
        cos_e = rowmax * (znorm * embn_e + 1e-6) * (
            1.0 / (SQRT2 * jnp.maximum(znorm, 1e-8)))
        cossum_scr[...] += jnp.sum(cos_e, keepdims=True) - cos_delta
        ecount_scr[...] += jnp.dot(
            ones_row, oh_e, preferred_element_type=jnp.float32) - hist_delta

    @pl.when(step == NSTEP - 1)
    def _fin():
        mean_cos = cossum_scr[...] / TOK
        loss_ref[...] = (1.0 - mean_cos) + BETA * (1.0 - mean_cos)
        e_mean = ecount_scr[...] / TOK
        losskl_ref[...] = -jnp.sum(
            e_mean * jnp.log((1.0 / N_E) / (e_mean + 1e-6)),
            axis=1, keepdims=True)
        perp_ref[...] = jnp.exp(
            -jnp.sum(e_mean * jnp.log(e_mean + 1e-6), axis=1, keepdims=True))


def kernel(z, embedding_weight):
    zf = z.reshape(TOK, E_DIM)
    out_shapes = (
        jax.ShapeDtypeStruct((TOK, N_E), jnp.float32),
        jax.ShapeDtypeStruct((TOK, N_E), jnp.float32),
        jax.ShapeDtypeStruct((TOK, N_E), jnp.float32),
        jax.ShapeDtypeStruct((NSTEP, 1, TILE), jnp.int32),
        jax.ShapeDtypeStruct((TOK, E_DIM), jnp.float32),
        jax.ShapeDtypeStruct((1, 1), jnp.float32),
        jax.ShapeDtypeStruct((1, 1), jnp.float32),
        jax.ShapeDtypeStruct((1, 1), jnp.float32),
    )
    big = pl.BlockSpec((TILE, N_E), lambda i: (i, 0))
    scalar = pl.BlockSpec((1, 1), lambda i: (0, 0))
    d, prob, onehot, idx, zq_st, loss, loss_kl, perp = pl.pallas_call(
        _vq_body,
        grid=(NSTEP,),
        in_specs=[
            pl.BlockSpec((TILE, E_DIM), lambda i: (i, 0)),
            pl.BlockSpec((N_E, E_DIM), lambda i: (0, 0)),
        ],
        out_specs=(
            big, big, big,
            pl.BlockSpec((1, 1, TILE), lambda i: (i, 0, 0)),
            pl.BlockSpec((TILE, E_DIM), lambda i: (i, 0)),
            scalar, scalar, scalar,
        ),
        out_shape=out_shapes,
        scratch_shapes=[
            pltpu.VMEM((N_E, AUG), jnp.float32),
            pltpu.VMEM((1, N_E), jnp.float32),
            pltpu.VMEM((1, N_E), jnp.float32),
            pltpu.VMEM((1, 1), jnp.float32),
        ],
    )(zf, embedding_weight)

    loss = loss[0, 0]
    loss_kl = loss_kl[0, 0]
    perplexity = perp[0, 0]
    constrative_loss = jnp.asarray(0.0, dtype=jnp.float32)
    z_q_st = zq_st.reshape(z.shape)
    idx = idx.reshape(TOK, 1)
    return (loss, constrative_loss, loss_kl, prob, d, z_q_st,
            perplexity, onehot, idx)
